# Initial kernel scaffold; baseline (speedup 1.0000x reference)
import jax
import jax.numpy as jnp
from jax.experimental import pallas as pl


def kernel(hidden_states, gate_w, up_w, down_w, fc1_w, fc2_w, expert_w, ln_g, ln_b):
    raise NotImplementedError("write your pallas kernel here")


# speedup vs baseline: 1.8497x; 1.8497x over previous
import functools

import jax
import jax.numpy as jnp
from jax.experimental import pallas as pl

_T = 32768
_H = 768
_I = 2048
_L = 256
_TILE = 1024
_EPS = 1e-5


def _fold_kernel(fc1_ref, ew_ref, fc2_ref, o_ref):
    a = jax.lax.dot_general(
        fc1_ref[...], ew_ref[...], (((0,), (0,)), ((), ())),
        preferred_element_type=jnp.float32)
    o_ref[...] = jax.lax.dot_general(
        a, fc2_ref[...], (((1,), (1,)), ((), ())),
        preferred_element_type=jnp.float32)


def _fused_kernel(x_ref, up_ref, down_ref, wlat_ref, g_ref, b_ref, o_ref):
    x = x_ref[...]
    h = jax.lax.dot_general(
        x, up_ref[...], (((1,), (1,)), ((), ())),
        preferred_element_type=jnp.float32)
    s = jnp.maximum(h, 0.0)
    s = s * s
    shared = jax.lax.dot_general(
        s, down_ref[...], (((1,), (1,)), ((), ())),
        preferred_element_type=jnp.float32)
    lat = jax.lax.dot_general(
        x, wlat_ref[...], (((1,), (0,)), ((), ())),
        preferred_element_type=jnp.float32)
    y = shared + lat
    mu = jnp.mean(y, axis=-1, keepdims=True)
    yc = y - mu
    var = jnp.mean(yc * yc, axis=-1, keepdims=True)
    o_ref[...] = yc * jax.lax.rsqrt(var + _EPS) * g_ref[...] + b_ref[...]


@functools.partial(jax.jit, static_argnames=())
def kernel(hidden_states, gate_w, up_w, down_w, fc1_w, fc2_w, expert_w, ln_g, ln_b):
    del gate_w

    w_lat = pl.pallas_call(
        _fold_kernel,
        out_shape=jax.ShapeDtypeStruct((_H, _H), jnp.float32),
    )(fc1_w, expert_w, fc2_w)

    grid = (_T // _TILE,)
    out = pl.pallas_call(
        _fused_kernel,
        grid=grid,
        in_specs=[
            pl.BlockSpec((_TILE, _H), lambda i: (i, 0)),
            pl.BlockSpec((_I, _H), lambda i: (0, 0)),
            pl.BlockSpec((_H, _I), lambda i: (0, 0)),
            pl.BlockSpec((_H, _H), lambda i: (0, 0)),
            pl.BlockSpec((1, _H), lambda i: (0, 0)),
            pl.BlockSpec((1, _H), lambda i: (0, 0)),
        ],
        out_specs=pl.BlockSpec((_TILE, _H), lambda i: (i, 0)),
        out_shape=jax.ShapeDtypeStruct((_T, _H), jnp.float32),
    )(hidden_states, up_w, down_w, w_lat,
      ln_g.reshape(1, _H), ln_b.reshape(1, _H))
    return out
